# baseline (speedup 1.0000x reference)
import functools

import numpy as np
import jax
import jax.numpy as jnp
from jax import lax
from jax.experimental import pallas as pl
from jax.experimental.pallas import tpu as pltpu
from jax.experimental.pallas import tpu_sc as plsc

_NF = 26
_D = 16
_B = 16384
_ROWS = _B * _NF
_FIELD = 100000
_ZDIM = _NF * _D


def _pair_row(i, j):
    return i * (_NF - 1) - i * (i - 1) // 2 + (j - i - 1)


_PR = []
for _d in range(1, _NF):
    for _i in range(_NF - _d):
        _PR.append(_pair_row(_i, _i + _d))
_GIDX = np.repeat(np.asarray(_PR, np.int32), _D)
_PDIM = _GIDX.shape[0]


_NC = 2
_NS = 16


@functools.cache
def _make_gather():
    nw = _NC * _NS
    rows_per_w = _ROWS // nw
    ch = 3328
    nchunk = rows_per_w // ch
    mesh = plsc.VectorSubcoreMesh(core_axis_name="c", subcore_axis_name="s")

    @functools.partial(
        pl.kernel,
        mesh=mesh,
        out_type=jax.ShapeDtypeStruct((_ROWS, _D), jnp.float32),
        scratch_types=[
            pltpu.VMEM((ch,), jnp.int32),
            pltpu.VMEM((ch, _D), jnp.float32),
            pltpu.SemaphoreType.DMA,
        ],
        compiler_params=pltpu.CompilerParams(use_tc_tiling_on_sc=False),
    )
    def gather_k(idx_hbm, table_hbm, out_hbm, idx_v, rows_v, sem):
        wid = lax.axis_index("s") * _NC + lax.axis_index("c")
        for c in range(nchunk):
            base = wid * rows_per_w + c * ch
            pltpu.sync_copy(idx_hbm.at[pl.ds(base, ch)], idx_v)
            pltpu.async_copy(table_hbm.at[idx_v], rows_v, sem).wait()
            pltpu.sync_copy(rows_v, out_hbm.at[pl.ds(base, ch)])

    return gather_k


def _mlp_body(emb_ref, bias_ref, w1z_ref, u_ref, b1_ref, w2_ref, b2_ref,
              w3t_ref, b3_ref, out_ref):
    e = emb_ref[...]
    zb = (e + bias_ref[...]).astype(jnp.bfloat16)
    acc = jnp.dot(zb, w1z_ref[...], preferred_element_type=jnp.float32)
    eb = e.astype(jnp.bfloat16)
    r = 0
    for d in range(1, _NF):
        w = (_NF - d) * _D
        prod = eb[:, :w] * eb[:, d * _D:]
        acc = acc + jnp.dot(prod, u_ref[r:r + w, :],
                            preferred_element_type=jnp.float32)
        r += w
    h1 = jnp.maximum(acc + b1_ref[...], 0.0).astype(jnp.bfloat16)
    h2 = jnp.maximum(
        jnp.dot(h1, w2_ref[...], preferred_element_type=jnp.float32)
        + b2_ref[...], 0.0)
    logit = jnp.sum(h2 * w3t_ref[...], axis=1, keepdims=True) + b3_ref[...]
    out_ref[...] = jax.nn.sigmoid(logit)


def kernel(x, W_emb, bias, W1, b1, W2, b2, W3, b3):
    offs = jnp.arange(_NF, dtype=jnp.int32) * _FIELD
    idx = (x + offs[None, :]).reshape(-1)

    emb = _make_gather()(idx, W_emb)
    emb2 = emb.reshape(_B, _ZDIM)

    w1z = W1[:_ZDIM].astype(jnp.bfloat16)
    u = W1[_ZDIM:][jnp.asarray(_GIDX)].astype(jnp.bfloat16)
    bias2 = bias.reshape(1, _ZDIM)
    b1r = b1.reshape(1, -1)
    b2r = b2.reshape(1, -1)
    w3t = W3.reshape(1, -1)
    b3r = b3.reshape(1, 1)

    tb = 512
    out = pl.pallas_call(
        _mlp_body,
        grid=(_B // tb,),
        in_specs=[
            pl.BlockSpec((tb, _ZDIM), lambda i: (i, 0)),
            pl.BlockSpec((1, _ZDIM), lambda i: (0, 0)),
            pl.BlockSpec((_ZDIM, 128), lambda i: (0, 0)),
            pl.BlockSpec((_PDIM, 128), lambda i: (0, 0)),
            pl.BlockSpec((1, 128), lambda i: (0, 0)),
            pl.BlockSpec((128, 64), lambda i: (0, 0)),
            pl.BlockSpec((1, 64), lambda i: (0, 0)),
            pl.BlockSpec((1, 64), lambda i: (0, 0)),
            pl.BlockSpec((1, 1), lambda i: (0, 0)),
        ],
        out_specs=pl.BlockSpec((tb, 1), lambda i: (i, 0)),
        out_shape=jax.ShapeDtypeStruct((_B, 1), jnp.float32),
    )(emb2, bias2, w1z, u, b1r, W2.astype(jnp.bfloat16), b2r,
      w3t, b3r)
    return out
